# Initial kernel scaffold; baseline (speedup 1.0000x reference)
import jax
import jax.numpy as jnp
from jax.experimental import pallas as pl


def kernel(x, row, col, aa):
    raise NotImplementedError("write your pallas kernel here")


# speedup vs baseline: 13.2470x; 13.2470x over previous
import functools

import jax
import jax.numpy as jnp
from jax import lax
from jax.experimental import pallas as pl
from jax.experimental.pallas import tpu as pltpu
from jax.experimental.pallas import tpu_sc as plsc

N = 10000
E = 320000
D = 128
H = 8
ALPHA = 0.2
L = 16
Q = E // 4
CHUNK = 2000
CSLICE = 2000


def _proj_body(w_ref, x_ref, o_ref):
    o_ref[...] = lax.dot_general(
        w_ref[...], x_ref[...], (((0,), (1,)), ((), ())),
        preferred_element_type=jnp.float32)


def _project(w, x):
    return pl.pallas_call(
        _proj_body,
        out_shape=jax.ShapeDtypeStruct((2 * H, N), jnp.float32),
    )(w, x)


_mesh = plsc.VectorSubcoreMesh(core_axis_name="c", subcore_axis_name="s")


@functools.partial(
    pl.kernel,
    out_type=jax.ShapeDtypeStruct((H * E,), jnp.float32),
    mesh=_mesh,
    compiler_params=pltpu.CompilerParams(needs_layout_passes=False),
    scratch_types=[
        pltpu.VMEM((N,), jnp.float32),
        pltpu.VMEM((N,), jnp.float32),
        pltpu.VMEM((N,), jnp.float32),
        pltpu.VMEM((N,), jnp.float32),
        pltpu.VMEM((CHUNK,), jnp.float32),
        pltpu.VMEM((CHUNK + 32,), jnp.int32),
        pltpu.VMEM((CHUNK,), jnp.int32),
        pltpu.VMEM((CSLICE,), jnp.float32),
        pltpu.VMEM_SHARED((16 * N,), jnp.float32),
    ],
)
def _sc_attn(sT_hbm, row_hbm, col_hbm, out_hbm,
             s1h, s2h, bst, acc, ebuf, rbuf, cbuf, tbuf, P):
    c = lax.axis_index("c")
    s = lax.axis_index("s")
    h = c * 4 + s // 4
    q = s % 4
    ebase = q * Q
    NV = CHUNK // L

    pltpu.sync_copy(sT_hbm.at[pl.ds(pl.multiple_of(h * N, 8), N)], s1h)
    pltpu.sync_copy(sT_hbm.at[pl.ds(pl.multiple_of((h + H) * N, 8), N)], s2h)

    zeros = jnp.zeros((L,), jnp.float32)
    lanes = lax.broadcasted_iota(jnp.int32, (L,), 0)
    lane0 = lanes == 0
    lane_last = lanes == L - 1

    def zbody(i, carry):
        bst[pl.ds(i * L, L)] = zeros
        acc[pl.ds(i * L, L)] = zeros
        return carry

    lax.fori_loop(0, N // L, zbody, 0)

    def chunk1(ci, carry):
        base = ebase + ci * CHUNK
        pltpu.sync_copy(row_hbm.at[pl.ds(base, CHUNK + 32)], rbuf)
        pltpu.sync_copy(col_hbm.at[pl.ds(base, CHUNK)], cbuf)

        def vbody(j, cin):
            off = j * L
            ridx = rbuf[pl.ds(off + 16, L)]
            rprev = plsc.load_gather(rbuf, [lanes + (off + 15)])
            rnext = plsc.load_gather(rbuf, [lanes + (off + 17)])
            cidx = cbuf[pl.ds(off, L)]
            v = plsc.load_gather(s1h, [ridx]) + plsc.load_gather(s2h, [cidx])
            v = jnp.maximum(v, ALPHA * v)
            ev = jnp.exp(v)
            ebuf[pl.ds(off, L)] = ev
            cs = plsc.cumsum(ev) + cin
            m_start = (ridx != rprev) | jnp.logical_and(
                lane0, jnp.logical_and(ci == 0, j == 0))
            m_end = (ridx != rnext) | jnp.logical_and(lane_last, j == NV - 1)
            plsc.store_scatter(bst, [ridx], cs - ev, mask=m_start)
            plsc.store_scatter(acc, [ridx], cs, mask=m_end)
            return cin + jnp.sum(ev)

        cout = lax.fori_loop(0, NV, vbody, carry)
        pltpu.sync_copy(
            ebuf, out_hbm.at[pl.ds(pl.multiple_of(h * E + base, 8), CHUNK)])
        return cout

    lax.fori_loop(0, Q // CHUNK, chunk1, 0.0)

    def dbody(i, carry):
        sl = pl.ds(i * L, L)
        acc[sl] = acc[sl] - bst[sl]
        return carry

    lax.fori_loop(0, N // L, dbody, 0)

    pltpu.sync_copy(acc, P.at[pl.ds(pl.multiple_of(s * N, 8), N)])
    plsc.subcore_barrier()

    hs = (s // 4) * 4

    def sib_loop(k, carry):
        sib = hs + lax.rem(q + 1 + k, 4)

        def cchunk(m, inner):
            nb = m * CSLICE
            pltpu.sync_copy(
                P.at[pl.ds(pl.multiple_of(sib * N + nb, 8), CSLICE)], tbuf)

            def addv(t, inner2):
                sl = pl.ds(nb + t * L, L)
                acc[sl] = acc[sl] + tbuf[pl.ds(t * L, L)]
                return inner2

            lax.fori_loop(0, CSLICE // L, addv, 0)
            return inner

        lax.fori_loop(0, N // CSLICE, cchunk, 0)
        return carry

    lax.fori_loop(0, 3, sib_loop, 0)

    def rbody(i, carry):
        sl = pl.ds(i * L, L)
        acc[sl] = 1.0 / (acc[sl] + 1e-12)
        return carry

    lax.fori_loop(0, N // L, rbody, 0)

    def chunk2(ci, carry):
        base = ebase + ci * CHUNK
        pltpu.sync_copy(row_hbm.at[pl.ds(base, CHUNK + 32)], rbuf)
        pltpu.sync_copy(
            out_hbm.at[pl.ds(pl.multiple_of(h * E + base, 8), CHUNK)], ebuf)

        def vbody2(j, inner):
            off = j * L
            ridx = rbuf[pl.ds(off + 16, L)]
            rs = plsc.load_gather(acc, [ridx])
            ebuf[pl.ds(off, L)] = ebuf[pl.ds(off, L)] * rs
            return inner

        lax.fori_loop(0, CHUNK // L, vbody2, 0)
        pltpu.sync_copy(
            ebuf, out_hbm.at[pl.ds(pl.multiple_of(h * E + base, 8), CHUNK)])
        return carry

    lax.fori_loop(0, Q // CHUNK, chunk2, 0)


@jax.jit
def kernel(x, row, col, aa):
    w = jnp.concatenate([aa[:, :D].T, aa[:, D:].T], axis=1)
    sT = _project(w, x).reshape(-1)
    row_pad = jnp.pad(row, (16, 16), constant_values=-1)
    return _sc_attn(sT, row_pad, col).reshape(H, E)


# speedup vs baseline: 13.7931x; 1.0412x over previous
import functools

import jax
import jax.numpy as jnp
from jax import lax
from jax.experimental import pallas as pl
from jax.experimental.pallas import tpu as pltpu
from jax.experimental.pallas import tpu_sc as plsc

N = 10000
E = 320000
D = 128
H = 8
ALPHA = 0.2
L = 16
Q = E // 4
CHUNK = 2000
CSLICE = 2000


def _proj_body(w_ref, x_ref, o_ref):
    o_ref[...] = lax.dot_general(
        w_ref[...], x_ref[...], (((0,), (1,)), ((), ())),
        preferred_element_type=jnp.float32)


def _project(w, x):
    return pl.pallas_call(
        _proj_body,
        out_shape=jax.ShapeDtypeStruct((2 * H, N), jnp.float32),
    )(w, x)


_mesh = plsc.VectorSubcoreMesh(core_axis_name="c", subcore_axis_name="s")


@functools.partial(
    pl.kernel,
    out_type=jax.ShapeDtypeStruct((H * E,), jnp.float32),
    mesh=_mesh,
    compiler_params=pltpu.CompilerParams(needs_layout_passes=False),
    scratch_types=[
        pltpu.VMEM((N,), jnp.float32),
        pltpu.VMEM((N,), jnp.float32),
        pltpu.VMEM((N,), jnp.float32),
        pltpu.VMEM((CHUNK,), jnp.float32),
        pltpu.VMEM((CHUNK + 32,), jnp.int32),
        pltpu.VMEM((CHUNK,), jnp.int32),
        pltpu.VMEM((CSLICE,), jnp.float32),
        pltpu.VMEM_SHARED((16 * N,), jnp.float32),
    ],
)
def _sc_attn(sT_hbm, row_hbm, col_hbm, out_hbm,
             s1h, s2h, acc, ebuf, rbuf, cbuf, tbuf, P):
    c = lax.axis_index("c")
    s = lax.axis_index("s")
    h = c * 4 + s // 4
    q = s % 4
    ebase = q * Q
    NV = CHUNK // L

    pltpu.sync_copy(sT_hbm.at[pl.ds(pl.multiple_of(h * N, 8), N)], s1h)
    pltpu.sync_copy(sT_hbm.at[pl.ds(pl.multiple_of((h + H) * N, 8), N)], s2h)

    zeros = jnp.zeros((L,), jnp.float32)
    lanes = lax.broadcasted_iota(jnp.int32, (L,), 0)
    lane0 = lanes == 0
    lane_last = lanes == L - 1

    def zbody(i, carry):
        acc[pl.ds(i * L, L)] = zeros
        return carry

    lax.fori_loop(0, N // L, zbody, 0)

    def chunk1(ci, carry):
        base = ebase + ci * CHUNK
        pltpu.sync_copy(row_hbm.at[pl.ds(base, CHUNK + 32)], rbuf)
        pltpu.sync_copy(col_hbm.at[pl.ds(base, CHUNK)], cbuf)

        def vbody(j, inner):
            off = j * L
            ridx = rbuf[pl.ds(off + 16, L)]
            rprev = plsc.load_gather(rbuf, [lanes + (off + 15)])
            rnext = plsc.load_gather(rbuf, [lanes + (off + 17)])
            cidx = cbuf[pl.ds(off, L)]
            v = plsc.load_gather(s1h, [ridx]) + plsc.load_gather(s2h, [cidx])
            v = jnp.maximum(v, ALPHA * v)
            ev = jnp.exp(v)
            ebuf[pl.ds(off, L)] = ev
            cs = plsc.cumsum(ev)
            m_start = (ridx != rprev) | lane0
            m_end = (ridx != rnext) | lane_last
            plsc.addupdate_scatter(acc, [ridx], ev - cs, mask=m_start)
            plsc.addupdate_scatter(acc, [ridx], cs, mask=m_end)
            return inner

        lax.fori_loop(0, NV, vbody, 0)
        pltpu.sync_copy(
            ebuf, out_hbm.at[pl.ds(pl.multiple_of(h * E + base, 8), CHUNK)])
        return carry

    lax.fori_loop(0, Q // CHUNK, chunk1, 0)

    pltpu.sync_copy(acc, P.at[pl.ds(pl.multiple_of(s * N, 8), N)])
    plsc.subcore_barrier()

    hs = (s // 4) * 4

    def sib_loop(k, carry):
        sib = hs + lax.rem(q + 1 + k, 4)

        def cchunk(m, inner):
            nb = m * CSLICE
            pltpu.sync_copy(
                P.at[pl.ds(pl.multiple_of(sib * N + nb, 8), CSLICE)], tbuf)

            def addv(t, inner2):
                sl = pl.ds(nb + t * L, L)
                acc[sl] = acc[sl] + tbuf[pl.ds(t * L, L)]
                return inner2

            lax.fori_loop(0, CSLICE // L, addv, 0)
            return inner

        lax.fori_loop(0, N // CSLICE, cchunk, 0)
        return carry

    lax.fori_loop(0, 3, sib_loop, 0)

    def rbody(i, carry):
        sl = pl.ds(i * L, L)
        acc[sl] = 1.0 / (acc[sl] + 1e-12)
        return carry

    lax.fori_loop(0, N // L, rbody, 0)

    def chunk2(ci, carry):
        base = ebase + ci * CHUNK
        pltpu.sync_copy(row_hbm.at[pl.ds(base, CHUNK + 32)], rbuf)
        pltpu.sync_copy(
            out_hbm.at[pl.ds(pl.multiple_of(h * E + base, 8), CHUNK)], ebuf)

        def vbody2(j, inner):
            off = j * L
            ridx = rbuf[pl.ds(off + 16, L)]
            rs = plsc.load_gather(acc, [ridx])
            ebuf[pl.ds(off, L)] = ebuf[pl.ds(off, L)] * rs
            return inner

        lax.fori_loop(0, CHUNK // L, vbody2, 0)
        pltpu.sync_copy(
            ebuf, out_hbm.at[pl.ds(pl.multiple_of(h * E + base, 8), CHUNK)])
        return carry

    lax.fori_loop(0, Q // CHUNK, chunk2, 0)


@jax.jit
def kernel(x, row, col, aa):
    w = jnp.concatenate([aa[:, :D].T, aa[:, D:].T], axis=1)
    sT = _project(w, x).reshape(-1)
    row_pad = jnp.pad(row, (16, 16), constant_values=-1)
    return _sc_attn(sT, row_pad, col).reshape(H, E)


# speedup vs baseline: 13.9036x; 1.0080x over previous
import functools

import jax
import jax.numpy as jnp
from jax import lax
from jax.experimental import pallas as pl
from jax.experimental.pallas import tpu as pltpu
from jax.experimental.pallas import tpu_sc as plsc

N = 10000
E = 320000
D = 128
H = 8
ALPHA = 0.2
L = 16
Q = E // 4
CHUNK = 2000
CSLICE = 2000
UNROLL = 5


def _proj_body(w_ref, x_ref, o_ref):
    o_ref[...] = lax.dot_general(
        w_ref[...], x_ref[...], (((0,), (1,)), ((), ())),
        preferred_element_type=jnp.float32)


def _project(w, x):
    return pl.pallas_call(
        _proj_body,
        out_shape=jax.ShapeDtypeStruct((2 * H, N), jnp.float32),
    )(w, x)


_mesh = plsc.VectorSubcoreMesh(core_axis_name="c", subcore_axis_name="s")


@functools.partial(
    pl.kernel,
    out_type=jax.ShapeDtypeStruct((H * E,), jnp.float32),
    mesh=_mesh,
    compiler_params=pltpu.CompilerParams(needs_layout_passes=False),
    scratch_types=[
        pltpu.VMEM((N,), jnp.float32),
        pltpu.VMEM((N,), jnp.float32),
        pltpu.VMEM((N,), jnp.float32),
        pltpu.VMEM((CHUNK,), jnp.float32),
        pltpu.VMEM((CHUNK + 32,), jnp.int32),
        pltpu.VMEM((CHUNK,), jnp.int32),
        pltpu.VMEM((CSLICE,), jnp.float32),
        pltpu.VMEM_SHARED((16 * N,), jnp.float32),
    ],
)
def _sc_attn(sT_hbm, row_hbm, col_hbm, out_hbm,
             s1h, s2h, acc, ebuf, rbuf, cbuf, tbuf, P):
    c = lax.axis_index("c")
    s = lax.axis_index("s")
    h = c * 4 + s // 4
    q = s % 4
    ebase = q * Q
    NV = CHUNK // L

    pltpu.sync_copy(sT_hbm.at[pl.ds(pl.multiple_of(h * N, 8), N)], s1h)
    pltpu.sync_copy(sT_hbm.at[pl.ds(pl.multiple_of((h + H) * N, 8), N)], s2h)

    zeros = jnp.zeros((L,), jnp.float32)
    lanes = lax.broadcasted_iota(jnp.int32, (L,), 0)
    lane0 = lanes == 0
    lane_last = lanes == L - 1

    def zbody(i, carry):
        acc[pl.ds(i * L, L)] = zeros
        return carry

    lax.fori_loop(0, N // L, zbody, 0)

    def chunk1(ci, carry):
        base = ebase + ci * CHUNK
        pltpu.sync_copy(row_hbm.at[pl.ds(base, CHUNK + 32)], rbuf)
        pltpu.sync_copy(col_hbm.at[pl.ds(base, CHUNK)], cbuf)

        def vbody(j, inner):
            for u in range(UNROLL):
                off = (j * UNROLL + u) * L
                ridx = rbuf[pl.ds(off + 16, L)]
                rprev = plsc.load_gather(rbuf, [lanes + (off + 15)])
                rnext = plsc.load_gather(rbuf, [lanes + (off + 17)])
                cidx = cbuf[pl.ds(off, L)]
                v = plsc.load_gather(s1h, [ridx]) + plsc.load_gather(s2h, [cidx])
                v = jnp.maximum(v, ALPHA * v)
                ev = jnp.exp(v)
                ebuf[pl.ds(off, L)] = ev
                cs = plsc.cumsum(ev)
                m_start = (ridx != rprev) | lane0
                m_end = (ridx != rnext) | lane_last
                plsc.addupdate_scatter(acc, [ridx], ev - cs, mask=m_start)
                plsc.addupdate_scatter(acc, [ridx], cs, mask=m_end)
            return inner

        lax.fori_loop(0, NV // UNROLL, vbody, 0)
        pltpu.sync_copy(
            ebuf, out_hbm.at[pl.ds(pl.multiple_of(h * E + base, 8), CHUNK)])
        return carry

    lax.fori_loop(0, Q // CHUNK, chunk1, 0)

    pltpu.sync_copy(acc, P.at[pl.ds(pl.multiple_of(s * N, 8), N)])
    plsc.subcore_barrier()

    hs = (s // 4) * 4

    def sib_loop(k, carry):
        sib = hs + lax.rem(q + 1 + k, 4)

        def cchunk(m, inner):
            nb = m * CSLICE
            pltpu.sync_copy(
                P.at[pl.ds(pl.multiple_of(sib * N + nb, 8), CSLICE)], tbuf)

            def addv(t, inner2):
                sl = pl.ds(nb + t * L, L)
                acc[sl] = acc[sl] + tbuf[pl.ds(t * L, L)]
                return inner2

            lax.fori_loop(0, CSLICE // L, addv, 0)
            return inner

        lax.fori_loop(0, N // CSLICE, cchunk, 0)
        return carry

    lax.fori_loop(0, 3, sib_loop, 0)

    def rbody(i, carry):
        sl = pl.ds(i * L, L)
        acc[sl] = 1.0 / (acc[sl] + 1e-12)
        return carry

    lax.fori_loop(0, N // L, rbody, 0)

    def chunk2(ci, carry):
        base = ebase + ci * CHUNK
        pltpu.sync_copy(row_hbm.at[pl.ds(base, CHUNK + 32)], rbuf)
        pltpu.sync_copy(
            out_hbm.at[pl.ds(pl.multiple_of(h * E + base, 8), CHUNK)], ebuf)

        def vbody2(j, inner):
            for u in range(UNROLL):
                off = (j * UNROLL + u) * L
                ridx = rbuf[pl.ds(off + 16, L)]
                rs = plsc.load_gather(acc, [ridx])
                ebuf[pl.ds(off, L)] = ebuf[pl.ds(off, L)] * rs
            return inner

        lax.fori_loop(0, NV // UNROLL, vbody2, 0)
        pltpu.sync_copy(
            ebuf, out_hbm.at[pl.ds(pl.multiple_of(h * E + base, 8), CHUNK)])
        return carry

    lax.fori_loop(0, Q // CHUNK, chunk2, 0)


@jax.jit
def kernel(x, row, col, aa):
    w = jnp.concatenate([aa[:, :D].T, aa[:, D:].T], axis=1)
    sT = _project(w, x).reshape(-1)
    row_pad = jnp.pad(row, (16, 16), constant_values=-1)
    return _sc_attn(sT, row_pad, col).reshape(H, E)


# speedup vs baseline: 20.4687x; 1.4722x over previous
import functools

import jax
import jax.numpy as jnp
from jax import lax
from jax.experimental import pallas as pl
from jax.experimental.pallas import tpu as pltpu
from jax.experimental.pallas import tpu_sc as plsc

N = 10000
E = 320000
D = 128
H = 8
ALPHA = 0.2
L = 16
Q = E // 4
CHUNK = 2000
NCH = Q // CHUNK
NV = CHUNK // L
UNROLL = 5
CSLICE = 2000


def _proj_body(w_ref, x_ref, o_ref):
    o_ref[...] = lax.dot_general(
        w_ref[...], x_ref[...], (((0,), (1,)), ((), ())),
        preferred_element_type=jnp.float32)


def _project(w, x):
    return pl.pallas_call(
        _proj_body,
        out_shape=jax.ShapeDtypeStruct((2 * H, N), jnp.float32),
    )(w, x)


_mesh = plsc.VectorSubcoreMesh(core_axis_name="c", subcore_axis_name="s")


@functools.partial(
    pl.kernel,
    out_type=jax.ShapeDtypeStruct((H * E,), jnp.float32),
    mesh=_mesh,
    compiler_params=pltpu.CompilerParams(needs_layout_passes=False),
    scratch_types=[
        pltpu.VMEM((N,), jnp.float32),
        pltpu.VMEM((N,), jnp.float32),
        pltpu.VMEM((N,), jnp.float32),
        pltpu.VMEM((CHUNK + 32,), jnp.int32),
        pltpu.VMEM((CHUNK + 32,), jnp.int32),
        pltpu.VMEM((CHUNK,), jnp.int32),
        pltpu.VMEM((CHUNK,), jnp.int32),
        pltpu.VMEM((CHUNK,), jnp.float32),
        pltpu.VMEM((CHUNK,), jnp.float32),
        pltpu.VMEM((CHUNK,), jnp.float32),
        pltpu.VMEM((CHUNK,), jnp.float32),
        pltpu.VMEM((CSLICE,), jnp.float32),
        pltpu.VMEM_SHARED((16 * N,), jnp.float32),
        pltpu.SemaphoreType.DMA,
        pltpu.SemaphoreType.DMA,
        pltpu.SemaphoreType.DMA,
        pltpu.SemaphoreType.DMA,
        pltpu.SemaphoreType.DMA,
        pltpu.SemaphoreType.DMA,
    ],
)
def _sc_attn(sT_hbm, row_hbm, col_hbm, out_hbm,
             s1h, s2h, acc, rbuf0, rbuf1, cbuf0, cbuf1, ebuf0, ebuf1,
             fbuf0, fbuf1, tbuf, P,
             rsem0, rsem1, csem0, csem1, osem0, osem1):
    c = lax.axis_index("c")
    s = lax.axis_index("s")
    h = c * 4 + s // 4
    q = s % 4
    ebase = q * Q

    rbuf = (rbuf0, rbuf1)
    cbuf = (cbuf0, cbuf1)
    ebuf = (ebuf0, ebuf1)
    fbuf = (fbuf0, fbuf1)
    rsem = (rsem0, rsem1)
    csem = (csem0, csem1)
    osem = (osem0, osem1)

    def rsrc(ci):
        return row_hbm.at[
            pl.ds(pl.multiple_of(ebase + ci * CHUNK, 8), CHUNK + 32)]

    def csrc(ci):
        return col_hbm.at[pl.ds(pl.multiple_of(ebase + ci * CHUNK, 8), CHUNK)]

    def ochunk(ci):
        return out_hbm.at[
            pl.ds(pl.multiple_of(h * E + ebase + ci * CHUNK, 8), CHUNK)]

    pltpu.sync_copy(sT_hbm.at[pl.ds(pl.multiple_of(h * N, 8), N)], s1h)
    pltpu.sync_copy(sT_hbm.at[pl.ds(pl.multiple_of((h + H) * N, 8), N)], s2h)

    zeros = jnp.zeros((L,), jnp.float32)
    lanes = lax.broadcasted_iota(jnp.int32, (L,), 0)
    lane0 = lanes == 0
    lane_last = lanes == L - 1

    def zbody(i, carry):
        acc[pl.ds(i * L, L)] = zeros
        return carry

    lax.fori_loop(0, N // L, zbody, 0)

    for b in range(2):
        pltpu.async_copy(rsrc(b), rbuf[b], rsem[b])
        pltpu.async_copy(csrc(b), cbuf[b], csem[b])

    def chunk1(ci2, carry):
        for b in range(2):
            ci = ci2 * 2 + b
            rb, cb, eb = rbuf[b], cbuf[b], ebuf[b]
            pltpu.make_async_copy(rsrc(ci), rb, rsem[b]).wait()
            pltpu.make_async_copy(csrc(ci), cb, csem[b]).wait()

            @pl.when(ci2 > 0)
            def _():
                pltpu.make_async_copy(eb, ochunk(ci - 2), osem[b]).wait()

            def vbody(j, inner):
                for u in range(UNROLL):
                    off = (j * UNROLL + u) * L
                    ridx = rb[pl.ds(off + 16, L)]
                    rprev = plsc.load_gather(rb, [lanes + (off + 15)])
                    rnext = plsc.load_gather(rb, [lanes + (off + 17)])
                    cidx = cb[pl.ds(off, L)]
                    v = (plsc.load_gather(s1h, [ridx])
                         + plsc.load_gather(s2h, [cidx]))
                    v = jnp.maximum(v, ALPHA * v)
                    ev = jnp.exp(v)
                    eb[pl.ds(off, L)] = ev
                    cs = plsc.cumsum(ev)
                    m_start = (ridx != rprev) | lane0
                    m_end = (ridx != rnext) | lane_last
                    plsc.addupdate_scatter(acc, [ridx], ev - cs, mask=m_start)
                    plsc.addupdate_scatter(acc, [ridx], cs, mask=m_end)
                return inner

            lax.fori_loop(0, NV // UNROLL, vbody, 0)
            pltpu.async_copy(eb, ochunk(ci), osem[b])

            @pl.when(ci + 2 < NCH)
            def _():
                pltpu.async_copy(rsrc(ci + 2), rb, rsem[b])
                pltpu.async_copy(csrc(ci + 2), cb, csem[b])

        return carry

    lax.fori_loop(0, NCH // 2, chunk1, 0)
    for b in range(2):
        pltpu.make_async_copy(ebuf[b], ochunk(NCH - 2 + b), osem[b]).wait()

    pltpu.sync_copy(acc, P.at[pl.ds(pl.multiple_of(s * N, 8), N)])
    plsc.subcore_barrier()

    hs = (s // 4) * 4

    def sib_loop(k, carry):
        sib = hs + lax.rem(q + 1 + k, 4)

        def cchunk(m, inner):
            nb = m * CSLICE
            pltpu.sync_copy(
                P.at[pl.ds(pl.multiple_of(sib * N + nb, 8), CSLICE)], tbuf)

            def addv(t, inner2):
                sl = pl.ds(nb + t * L, L)
                acc[sl] = acc[sl] + tbuf[pl.ds(t * L, L)]
                return inner2

            lax.fori_loop(0, CSLICE // L, addv, 0)
            return inner

        lax.fori_loop(0, N // CSLICE, cchunk, 0)
        return carry

    lax.fori_loop(0, 3, sib_loop, 0)

    def rbody(i, carry):
        sl = pl.ds(i * L, L)
        acc[sl] = 1.0 / (acc[sl] + 1e-12)
        return carry

    lax.fori_loop(0, N // L, rbody, 0)

    for b in range(2):
        pltpu.async_copy(rsrc(b), rbuf[b], rsem[b])
        pltpu.async_copy(ochunk(b), fbuf[b], csem[b])

    def chunk2(ci2, carry):
        for b in range(2):
            ci = ci2 * 2 + b
            rb, fb, eb = rbuf[b], fbuf[b], ebuf[b]
            pltpu.make_async_copy(rsrc(ci), rb, rsem[b]).wait()
            pltpu.make_async_copy(ochunk(ci), fb, csem[b]).wait()

            @pl.when(ci2 > 0)
            def _():
                pltpu.make_async_copy(eb, ochunk(ci - 2), osem[b]).wait()

            def vbody2(j, inner):
                for u in range(UNROLL):
                    off = (j * UNROLL + u) * L
                    ridx = rb[pl.ds(off + 16, L)]
                    rs = plsc.load_gather(acc, [ridx])
                    eb[pl.ds(off, L)] = fb[pl.ds(off, L)] * rs
                return inner

            lax.fori_loop(0, NV // UNROLL, vbody2, 0)
            pltpu.async_copy(eb, ochunk(ci), osem[b])

            @pl.when(ci + 2 < NCH)
            def _():
                pltpu.async_copy(rsrc(ci + 2), rb, rsem[b])
                pltpu.async_copy(ochunk(ci + 2), fb, csem[b])

        return carry

    lax.fori_loop(0, NCH // 2, chunk2, 0)
    for b in range(2):
        pltpu.make_async_copy(ebuf[b], ochunk(NCH - 2 + b), osem[b]).wait()


@jax.jit
def kernel(x, row, col, aa):
    w = jnp.concatenate([aa[:, :D].T, aa[:, D:].T], axis=1)
    sT = _project(w, x).reshape(-1)
    row_pad = jnp.pad(row, (16, 16), constant_values=-1)
    return _sc_attn(sT, row_pad, col).reshape(H, E)
